# baseline (speedup 1.0000x reference)
import jax
import jax.numpy as jnp
from jax import lax
from jax.experimental import pallas as pl
from jax.experimental.pallas import tpu as pltpu
from jax.experimental.pallas import tpu_sc as plsc

BATCH = 16384
EMB = 64
NROWS = 1000000
NC = 2
NS = 16
NW = NC * NS
BPW = BATCH // NW
LANES = 16
GRAN_ROWS = NROWS // LANES
CH = LANES
ROWS_PER_CH = EMB * CH


def _body(p_hbm, q_hbm, uid_hbm, bid_hbm, out_hbm,
          uidx_v, bidx_v, idx_u, idx_q, gat_u, gat_q, out_v, sem_u, sem_q):
    wid = lax.axis_index("s") * NC + lax.axis_index("c")
    base = wid * BPW
    pltpu.sync_copy(uid_hbm.at[pl.ds(base, BPW)], uidx_v)
    pltpu.sync_copy(bid_hbm.at[pl.ds(base, BPW)], bidx_v)
    lanes = lax.iota(jnp.int32, LANES)

    def chunk(c, _):
        uvec = uidx_v[pl.ds(c * CH, CH)]
        bvec = bidx_v[pl.ds(c * CH, CH)]
        ug = jnp.right_shift(uvec, 4)
        bg = jnp.right_shift(bvec, 4)
        for d in range(EMB):
            idx_u[pl.ds(d * CH, CH)] = ug + d * GRAN_ROWS
            idx_q[pl.ds(d * CH, CH)] = bg + d * GRAN_ROWS
        cu = pltpu.async_copy(p_hbm.at[idx_u], gat_u, sem_u)
        cq = pltpu.async_copy(q_hbm.at[idx_q], gat_q, sem_q)
        cu.wait()
        cq.wait()

        ul = jnp.bitwise_and(uvec, 15)
        bl = jnp.bitwise_and(bvec, 15)
        acc = jnp.zeros((LANES,), jnp.float32)
        for d in range(EMB):
            rows = jnp.full((LANES,), d * CH, jnp.int32) + lanes
            u = plsc.load_gather(gat_u, [rows, ul])
            q = plsc.load_gather(gat_q, [rows, bl])
            acc = acc + u * q
        out_v[pl.ds(c * CH, CH)] = acc
        return 0

    lax.fori_loop(0, BPW // CH, chunk, 0)
    pltpu.sync_copy(out_v, out_hbm.at[pl.ds(base, BPW)])


_sc_call = pl.kernel(
    _body,
    out_type=jax.ShapeDtypeStruct((BATCH,), jnp.float32),
    mesh=plsc.VectorSubcoreMesh(
        core_axis_name="c", subcore_axis_name="s",
        num_cores=NC, num_subcores=NS),
    scratch_types=[
        pltpu.VMEM((BPW,), jnp.int32),
        pltpu.VMEM((BPW,), jnp.int32),
        pltpu.VMEM((ROWS_PER_CH,), jnp.int32),
        pltpu.VMEM((ROWS_PER_CH,), jnp.int32),
        pltpu.VMEM((ROWS_PER_CH, LANES), jnp.float32),
        pltpu.VMEM((ROWS_PER_CH, LANES), jnp.float32),
        pltpu.VMEM((BPW,), jnp.float32),
        pltpu.SemaphoreType.DMA,
        pltpu.SemaphoreType.DMA,
    ],
    compiler_params=pltpu.CompilerParams(
        needs_layout_passes=False, use_tc_tiling_on_sc=False),
)


@jax.jit
def kernel(P, Q, user_id, book_id):
    pf = P.T.reshape(EMB * GRAN_ROWS, LANES)
    qf = Q.T.reshape(EMB * GRAN_ROWS, LANES)
    return _sc_call(pf, qf,
                    user_id.astype(jnp.int32), book_id.astype(jnp.int32))


# speedup vs baseline: 1.0027x; 1.0027x over previous
import jax
import jax.numpy as jnp
from jax import lax
from jax.experimental import pallas as pl
from jax.experimental.pallas import tpu as pltpu
from jax.experimental.pallas import tpu_sc as plsc

BATCH = 16384
EMB = 64
NROWS = 1000000
NC = 2
NS = 16
NW = NC * NS
BPW = BATCH // NW
LANES = 16


def _body(p_hbm, q_hbm, uid_hbm, bid_hbm, out_hbm,
          uidx_v, bidx_v, gat_u, gat_q, out_v, sem_u, sem_q):
    wid = lax.axis_index("s") * NC + lax.axis_index("c")
    base = wid * BPW
    pltpu.sync_copy(uid_hbm.at[pl.ds(base, BPW)], uidx_v)
    pltpu.sync_copy(bid_hbm.at[pl.ds(base, BPW)], bidx_v)

    def group(g, _):
        uvec = uidx_v[pl.ds(g * LANES, LANES)]
        bvec = bidx_v[pl.ds(g * LANES, LANES)]
        cps = []
        for d in range(EMB):
            cps.append(pltpu.async_copy(
                p_hbm.at[uvec + d * NROWS],
                gat_u.at[pl.ds(d * LANES, LANES)], sem_u))
            cps.append(pltpu.async_copy(
                q_hbm.at[bvec + d * NROWS],
                gat_q.at[pl.ds(d * LANES, LANES)], sem_q))
        for cp in cps:
            cp.wait()

        acc = jnp.zeros((LANES,), jnp.float32)
        for d in range(EMB):
            acc = acc + gat_u[pl.ds(d * LANES, LANES)] * \
                gat_q[pl.ds(d * LANES, LANES)]
        out_v[pl.ds(g * LANES, LANES)] = acc
        return 0

    lax.fori_loop(0, BPW // LANES, group, 0)
    pltpu.sync_copy(out_v, out_hbm.at[pl.ds(base, BPW)])


_sc_call = pl.kernel(
    _body,
    out_type=jax.ShapeDtypeStruct((BATCH,), jnp.float32),
    mesh=plsc.VectorSubcoreMesh(
        core_axis_name="c", subcore_axis_name="s",
        num_cores=NC, num_subcores=NS),
    scratch_types=[
        pltpu.VMEM((BPW,), jnp.int32),
        pltpu.VMEM((BPW,), jnp.int32),
        pltpu.VMEM((EMB * LANES,), jnp.float32),
        pltpu.VMEM((EMB * LANES,), jnp.float32),
        pltpu.VMEM((BPW,), jnp.float32),
        pltpu.SemaphoreType.DMA,
        pltpu.SemaphoreType.DMA,
    ],
    compiler_params=pltpu.CompilerParams(
        needs_layout_passes=False, use_tc_tiling_on_sc=False),
)


@jax.jit
def kernel(P, Q, user_id, book_id):
    pf = P.T.reshape(NROWS * EMB)
    qf = Q.T.reshape(NROWS * EMB)
    return _sc_call(pf, qf,
                    user_id.astype(jnp.int32), book_id.astype(jnp.int32))


# speedup vs baseline: 9.1686x; 9.1441x over previous
import jax
import jax.numpy as jnp
from jax import lax
from jax.experimental import pallas as pl
from jax.experimental.pallas import tpu as pltpu
from jax.experimental.pallas import tpu_sc as plsc

BATCH = 16384
EMB = 64
NC = 2
NS = 16
NW = NC * NS
BPW = BATCH // NW
LANES = 16


def _body(p_hbm, q_hbm, uid_hbm, bid_hbm, out_hbm,
          uidx_v, bidx_v, urows_v, qrows_v, out_v, sem_u, sem_q):
    wid = lax.axis_index("s") * NC + lax.axis_index("c")
    base = wid * BPW

    pltpu.sync_copy(uid_hbm.at[pl.ds(base, BPW)], uidx_v)
    pltpu.sync_copy(bid_hbm.at[pl.ds(base, BPW)], bidx_v)

    cp_u = pltpu.async_copy(p_hbm.at[uidx_v], urows_v, sem_u)
    cp_q = pltpu.async_copy(q_hbm.at[bidx_v], qrows_v, sem_q)
    cp_u.wait()
    cp_q.wait()

    lanes = lax.iota(jnp.int32, LANES)

    def group(g, _):
        vec = jnp.zeros((LANES,), jnp.float32)
        for j in range(LANES):
            r = g * LANES + j
            acc = urows_v[r, pl.ds(0, LANES)] * qrows_v[r, pl.ds(0, LANES)]
            for k in range(1, EMB // LANES):
                acc = acc + urows_v[r, pl.ds(k * LANES, LANES)] * \
                    qrows_v[r, pl.ds(k * LANES, LANES)]
            vec = jnp.where(lanes == j, jnp.sum(acc), vec)
        out_v[pl.ds(g * LANES, LANES)] = vec
        return 0

    lax.fori_loop(0, BPW // LANES, group, 0)

    pltpu.sync_copy(out_v, out_hbm.at[pl.ds(base, BPW)])


_sc_call = pl.kernel(
    _body,
    out_type=jax.ShapeDtypeStruct((BATCH,), jnp.float32),
    mesh=plsc.VectorSubcoreMesh(
        core_axis_name="c", subcore_axis_name="s",
        num_cores=NC, num_subcores=NS),
    scratch_types=[
        pltpu.VMEM((BPW,), jnp.int32),
        pltpu.VMEM((BPW,), jnp.int32),
        pltpu.VMEM((BPW, EMB), jnp.float32),
        pltpu.VMEM((BPW, EMB), jnp.float32),
        pltpu.VMEM((BPW,), jnp.float32),
        pltpu.SemaphoreType.DMA,
        pltpu.SemaphoreType.DMA,
    ],
    compiler_params=pltpu.CompilerParams(
        needs_layout_passes=False, use_tc_tiling_on_sc=False),
)


@jax.jit
def kernel(P, Q, user_id, book_id):
    return _sc_call(P, Q, user_id.astype(jnp.int32), book_id.astype(jnp.int32))
